# Initial kernel scaffold; baseline (speedup 1.0000x reference)
import jax
import jax.numpy as jnp
from jax.experimental import pallas as pl


def kernel(x, edge_index, edge_weight, W1, b1, W2, b2, W3, b3):
    raise NotImplementedError("write your pallas kernel here")


# speedup vs baseline: 4.8054x; 4.8054x over previous
import functools

import jax
import jax.numpy as jnp
from jax import lax
from jax.experimental import pallas as pl
from jax.experimental.pallas import tpu as pltpu
from jax.experimental.pallas import tpu_sc as plsc

N = 10000
F = 128
E = 320000
NC = 2
NS = 16
NW = NC * NS
EPT = E // NW
CHUNK = 80
NCHUNK = EPT // CHUNK
ROWS_PT = 632
NP = NS * ROWS_PT
LANES = 16


def _spmm_body(support, src, dst, w, zeros, out,
               src_v, dst_v, w_v, rows_v, acc, sem):
    c = lax.axis_index("c")
    s = lax.axis_index("s")
    wid = c * NS + s

    pltpu.sync_copy(zeros, acc.at[pl.ds(s * ROWS_PT, ROWS_PT)])
    pltpu.sync_copy(src.at[pl.ds(wid * EPT, EPT)], src_v)
    pltpu.sync_copy(w.at[pl.ds(wid * EPT, EPT)], w_v)
    plsc.subcore_barrier()

    def g_body(g, carry):
        pltpu.async_copy(support.at[src_v.at[pl.ds(g * CHUNK, CHUNK)]],
                         rows_v, sem).wait()
        pltpu.sync_copy(dst.at[pl.ds(wid * EPT + g * CHUNK, CHUNK)], dst_v)

        def e_body(e16, carry2):
            wv = w_v[pl.ds(g * CHUNK + e16 * LANES, LANES)]
            for j in range(LANES):
                we = wv[j]
                e = e16 * LANES + j
                for f in range(F // LANES):
                    sl = pl.ds(f * LANES, LANES)
                    rows_v[e, sl] = rows_v[e, sl] * we
            return carry2

        lax.fori_loop(0, CHUNK // LANES, e_body, 0)
        pltpu.sync_copy(rows_v, acc.at[dst_v], add=True)
        return carry

    lax.fori_loop(0, NCHUNK, g_body, 0)
    plsc.subcore_barrier()
    pltpu.sync_copy(acc.at[pl.ds(s * ROWS_PT, ROWS_PT)],
                    out.at[c, pl.ds(s * ROWS_PT, ROWS_PT)])


_spmm = functools.partial(
    pl.kernel,
    out_type=jax.ShapeDtypeStruct((NC, NP, F), jnp.float32),
    mesh=plsc.VectorSubcoreMesh(core_axis_name="c", subcore_axis_name="s"),
    scratch_types=[
        pltpu.VMEM((EPT,), jnp.int32),
        pltpu.VMEM((CHUNK,), jnp.int32),
        pltpu.VMEM((EPT,), jnp.float32),
        pltpu.VMEM((CHUNK, F), jnp.float32),
        pltpu.VMEM_SHARED((NP, F), jnp.float32),
        pltpu.SemaphoreType.DMA,
    ],
)(_spmm_body)


BM = 1000


def _mm_body(x_ref, w_ref, o_ref):
    o_ref[...] = jnp.dot(x_ref[...], w_ref[...],
                         preferred_element_type=jnp.float32)


def _matmul(x, W):
    return pl.pallas_call(
        _mm_body,
        grid=(N // BM,),
        in_specs=[pl.BlockSpec((BM, F), lambda i: (i, 0)),
                  pl.BlockSpec((F, F), lambda i: (0, 0))],
        out_specs=pl.BlockSpec((BM, F), lambda i: (i, 0)),
        out_shape=jax.ShapeDtypeStruct((N, F), jnp.float32),
    )(x, W)


def _prologue(p_ref, b_ref):
    return jnp.maximum(p_ref[0] + p_ref[1] + b_ref[...], 0.0)


def _supp_body(p_ref, b_ref, w_ref, s_ref):
    s_ref[...] = jnp.dot(_prologue(p_ref, b_ref), w_ref[...],
                         preferred_element_type=jnp.float32)


def _fuse_body(p_ref, b_ref, w_ref, h_ref, s_ref):
    h = _prologue(p_ref, b_ref)
    h_ref[...] = h
    s_ref[...] = jnp.dot(h, w_ref[...], preferred_element_type=jnp.float32)


def _final_body(p_ref, b_ref, h_ref):
    h_ref[...] = _prologue(p_ref, b_ref)


_P_SPEC = pl.BlockSpec((NC, BM, F), lambda i: (0, i, 0))
_B_SPEC = pl.BlockSpec((1, F), lambda i: (0, 0))
_W_SPEC = pl.BlockSpec((F, F), lambda i: (0, 0))
_H_SPEC = pl.BlockSpec((BM, F), lambda i: (i, 0))
_HS = jax.ShapeDtypeStruct((N, F), jnp.float32)


def _support_only(p, b, W):
    return pl.pallas_call(
        _supp_body, grid=(N // BM,),
        in_specs=[_P_SPEC, _B_SPEC, _W_SPEC],
        out_specs=_H_SPEC, out_shape=_HS,
    )(p, b, W)


def _fuse(p, b, W):
    return pl.pallas_call(
        _fuse_body, grid=(N // BM,),
        in_specs=[_P_SPEC, _B_SPEC, _W_SPEC],
        out_specs=(_H_SPEC, _H_SPEC), out_shape=(_HS, _HS),
    )(p, b, W)


def _final(p, b):
    return pl.pallas_call(
        _final_body, grid=(N // BM,),
        in_specs=[_P_SPEC, _B_SPEC],
        out_specs=_H_SPEC, out_shape=_HS,
    )(p, b)


def kernel(x, edge_index, edge_weight, W1, b1, W2, b2, W3, b3):
    src = edge_index[0]
    dst = edge_index[1]
    w2d = edge_weight
    zeros = jnp.zeros((ROWS_PT, F), jnp.float32)
    b1r, b2r, b3r = (b.reshape(1, F) for b in (b1, b2, b3))

    t = _matmul(x, W1)
    p = _spmm(t, src, dst, w2d, zeros)
    t = _support_only(p, b1r, W2)
    p = _spmm(t, src, dst, w2d, zeros)
    h2, t = _fuse(p, b2r, W3)
    p = _spmm(t, src, dst, w2d, zeros)
    h3, t = _fuse(p, b3r, W3)
    p = _spmm(t, src, dst, w2d, zeros)
    h4, t = _fuse(p, b3r, W3)
    p = _spmm(t, src, dst, w2d, zeros)
    h5 = _final(p, b3r)
    return (h2, h3, h4, h5)
